# Initial kernel scaffold; baseline (speedup 1.0000x reference)
import jax
import jax.numpy as jnp
from jax.experimental import pallas as pl


def kernel(embeddings, positional_embeddings, edge_index, batch, W_pre, b_pre, W_g1, b_g1, W_g2, b_g2, W_head, b_head):
    raise NotImplementedError("write your pallas kernel here")


# speedup vs baseline: 8.8568x; 8.8568x over previous
import functools
import jax
import jax.numpy as jnp
from jax import lax
from jax.experimental import pallas as pl
from jax.experimental.pallas import tpu as pltpu
from jax.experimental.pallas import tpu_sc as plsc

NC = 2
NS = 16
NW = NC * NS
K = 128

_F32 = jnp.float32
_HI = lax.Precision.HIGHEST


def _dot(a, b):
    return lax.dot_general(a, b, (((a.ndim - 1,), (0,)), ((), ())),
                           precision=_HI, preferred_element_type=_F32)


def _dott(a, b):
    return lax.dot_general(a, b, (((0,), (0,)), ((), ())),
                           precision=_HI, preferred_element_type=_F32)


def _sc_mesh():
    return plsc.VectorSubcoreMesh(core_axis_name="c", subcore_axis_name="s",
                                  num_cores=NC, num_subcores=NS)


def _zero_acc(zrows, acc, s, rows, width):
    base = s * rows
    nfull = rows // K
    for j in range(nfull):
        pltpu.sync_copy(zrows, acc.at[pl.ds(base + j * K, K)])
    rem = rows - nfull * K
    if rem:
        pltpu.sync_copy(zrows.at[pl.ds(0, rem)],
                        acc.at[pl.ds(base + nfull * K, rem)])


def _make_deg_kernel(npad, nch):
    rows = npad // NS

    @functools.partial(
        pl.kernel,
        out_type=jax.ShapeDtypeStruct((NC, npad, 128), _F32),
        mesh=_sc_mesh(),
        scratch_types=[
            pltpu.VMEM((nch, K), jnp.int32),
            pltpu.VMEM((K, 128), _F32),
            pltpu.VMEM_SHARED((npad, 128), _F32),
        ],
    )
    def deg_kernel(dstw, ones_hbm, zeros_hbm, out, idx_v, buf, acc):
        c = lax.axis_index("c")
        s = lax.axis_index("s")
        wid = c * NS + s
        pltpu.sync_copy(dstw.at[wid], idx_v)
        pltpu.sync_copy(zeros_hbm, buf)
        _zero_acc(buf, acc, s, rows, 128)
        plsc.subcore_barrier()
        pltpu.sync_copy(ones_hbm, buf)

        def body(t, carry):
            pltpu.sync_copy(buf, acc.at[idx_v.at[t]], add=True)
            return carry

        lax.fori_loop(0, nch, body, 0)
        plsc.subcore_barrier()
        pltpu.sync_copy(acc.at[pl.ds(s * rows, rows)],
                        out.at[c, pl.ds(s * rows, rows)])

    return deg_kernel


def _make_gs_kernel(n, npad, nch):
    rows = npad // NS
    nph = 2
    nch2 = nch // nph

    @functools.partial(
        pl.kernel,
        out_type=jax.ShapeDtypeStruct((NC, npad, 128), _F32),
        mesh=_sc_mesh(),
        scratch_types=[
            pltpu.VMEM((nch2, K), jnp.int32),
            pltpu.VMEM((nch2, K), jnp.int32),
            pltpu.VMEM((K, 128), _F32),
            pltpu.VMEM((K, 128), _F32),
            pltpu.VMEM_SHARED((npad, 128), _F32),
            pltpu.SemaphoreType.DMA,
            pltpu.SemaphoreType.DMA,
        ],
    )
    def gs_kernel(y, srcw, dstw, zeros_hbm, out,
                  sidx, didx, b0, b1, acc, sem0, sem1):
        c = lax.axis_index("c")
        s = lax.axis_index("s")
        wid = c * NS + s
        pltpu.sync_copy(zeros_hbm, b0)
        _zero_acc(b0, acc, s, rows, 128)
        plsc.subcore_barrier()

        def body(i, carry):
            t0 = 2 * i
            cp0 = pltpu.async_copy(y.at[sidx.at[t0]], b0, sem0)
            cp1 = pltpu.async_copy(y.at[sidx.at[t0 + 1]], b1, sem1)
            cp0.wait()
            pltpu.sync_copy(b0, acc.at[didx.at[t0]], add=True)
            cp1.wait()
            pltpu.sync_copy(b1, acc.at[didx.at[t0 + 1]], add=True)
            return carry

        for p in range(nph):
            pltpu.sync_copy(srcw.at[wid, pl.ds(p * nch2, nch2)], sidx)
            pltpu.sync_copy(dstw.at[wid, pl.ds(p * nch2, nch2)], didx)
            lax.fori_loop(0, nch2 // 2, body, 0)
        plsc.subcore_barrier()
        pltpu.sync_copy(acc.at[pl.ds(s * rows, rows)],
                        out.at[c, pl.ds(s * rows, rows)])

    return gs_kernel


def _pre_body(emb, pos, wpre, bpre, wg1, deg, y1):
    n = y1.shape[0]
    degsum = jnp.sum(deg[0, :n, :] + deg[1, :n, :], axis=1,
                     keepdims=True) + 1.0
    dis = lax.rsqrt(degsum)
    x = jnp.maximum(_dot(emb[...] + pos[...], wpre[...]) + bpre[...], 0.0)
    y1[...] = dis * _dot(x, wg1[...])


def _mid_body(acc, y, deg, bg, wg2, y2):
    n = y.shape[0]
    degsum = jnp.sum(deg[0, :n, :] + deg[1, :n, :], axis=1,
                     keepdims=True) + 1.0
    dis = lax.rsqrt(degsum)
    h = jnp.maximum(dis * (acc[0, :n, :] + acc[1, :n, :] + y[...])
                    + bg[...], 0.0)
    y2[...] = dis * _dot(h, wg2[...])


def _post_body(acc, y, deg, bg, batch2, wh, bh, out):
    n = y.shape[0]
    nseg = out.shape[0]
    degsum = jnp.sum(deg[0, :n, :] + deg[1, :n, :], axis=1,
                     keepdims=True) + 1.0
    dis = lax.rsqrt(degsum)
    h = jnp.maximum(dis * (acc[0, :n, :] + acc[1, :n, :] + y[...])
                    + bg[...], 0.0)
    seg_ids = lax.broadcasted_iota(jnp.int32, (1, nseg), 1)
    onehot = (batch2[...] == seg_ids).astype(_F32)
    seg = _dott(onehot, h)
    cnt = _dott(onehot, jnp.ones((n, 1), _F32))
    pooled = seg / jnp.maximum(cnt, 1.0)
    out[...] = jnp.maximum(_dot(pooled, wh[...]) + bh[...], 0.0)


def kernel(embeddings, positional_embeddings, edge_index, batch,
           W_pre, b_pre, W_g1, b_g1, W_g2, b_g2, W_head, b_head):
    n, d = embeddings.shape
    e = edge_index.shape[1]

    epw = -(-e // NW)
    nch = -(-(-(-epw // K)) // 4) * 4
    ep = nch * K
    npad = -(-(n + 1) // (NS * 8)) * (NS * 8)

    src = edge_index[0]
    dst = edge_index[1]
    pad_total = NW * ep - e
    src_p = jnp.pad(src, (0, pad_total)).reshape(NW, nch, K)
    dst_p = jnp.pad(dst, (0, pad_total),
                    constant_values=n).reshape(NW, nch, K)

    ones128 = jnp.zeros((K, d), _F32).at[:, 0].set(1.0)
    zeros128 = jnp.zeros((K, d), _F32)

    deg = _make_deg_kernel(npad, nch)(dst_p, ones128, zeros128)

    gs = _make_gs_kernel(n, npad, nch)

    y1 = pl.pallas_call(
        _pre_body,
        out_shape=jax.ShapeDtypeStruct((n, d), _F32),
    )(embeddings, positional_embeddings, W_pre, b_pre.reshape(1, d),
      W_g1, deg)

    acc1 = gs(y1, src_p, dst_p, zeros128)

    y2 = pl.pallas_call(
        _mid_body,
        out_shape=jax.ShapeDtypeStruct((n, d), _F32),
    )(acc1, y1, deg, b_g1.reshape(1, d), W_g2)

    acc2 = gs(y2, src_p, dst_p, zeros128)

    d_out = W_head.shape[1]
    out = pl.pallas_call(
        _post_body,
        out_shape=jax.ShapeDtypeStruct((16, d_out), _F32),
    )(acc2, y2, deg, b_g2.reshape(1, d), batch.reshape(n, 1),
      W_head, b_head.reshape(1, d_out))
    return out


# speedup vs baseline: 19.5655x; 2.2091x over previous
import functools
import jax
import jax.numpy as jnp
from jax import lax
from jax.experimental import pallas as pl
from jax.experimental.pallas import tpu as pltpu
from jax.experimental.pallas import tpu_sc as plsc

NC = 2
NS = 16
NW = NC * NS
K = 128

_F32 = jnp.float32
_HI = lax.Precision.HIGHEST


def _dot(a, b):
    return lax.dot_general(a, b, (((a.ndim - 1,), (0,)), ((), ())),
                           precision=_HI, preferred_element_type=_F32)


def _dott(a, b):
    return lax.dot_general(a, b, (((0,), (0,)), ((), ())),
                           precision=_HI, preferred_element_type=_F32)


def _sc_mesh():
    return plsc.VectorSubcoreMesh(core_axis_name="c", subcore_axis_name="s",
                                  num_cores=NC, num_subcores=NS)


def _zero_acc(zrows, acc, s, rows, width):
    base = s * rows
    nfull = rows // K
    for j in range(nfull):
        pltpu.sync_copy(zrows, acc.at[pl.ds(base + j * K, K)])
    rem = rows - nfull * K
    if rem:
        pltpu.sync_copy(zrows.at[pl.ds(0, rem)],
                        acc.at[pl.ds(base + nfull * K, rem)])


def _make_deg_kernel(npad, nch):
    rows = npad // NS

    @functools.partial(
        pl.kernel,
        out_type=jax.ShapeDtypeStruct((NC, npad, 128), _F32),
        mesh=_sc_mesh(),
        scratch_types=[
            pltpu.VMEM((nch, K), jnp.int32),
            pltpu.VMEM((K, 128), _F32),
            pltpu.VMEM_SHARED((npad, 128), _F32),
        ],
    )
    def deg_kernel(dstw, ones_hbm, zeros_hbm, out, idx_v, buf, acc):
        c = lax.axis_index("c")
        s = lax.axis_index("s")
        wid = c * NS + s
        pltpu.sync_copy(dstw.at[wid], idx_v)
        pltpu.sync_copy(zeros_hbm, buf)
        _zero_acc(buf, acc, s, rows, 128)
        plsc.subcore_barrier()
        pltpu.sync_copy(ones_hbm, buf)

        def body(t, carry):
            pltpu.sync_copy(buf, acc.at[idx_v.at[t]], add=True)
            return carry

        lax.fori_loop(0, nch, body, 0)
        plsc.subcore_barrier()
        pltpu.sync_copy(acc.at[pl.ds(s * rows, rows)],
                        out.at[c, pl.ds(s * rows, rows)])

    return deg_kernel


def _make_gs_kernel(n, npad, nch):
    rows = npad // NS
    nph = 2
    nch2 = nch // nph

    @functools.partial(
        pl.kernel,
        out_type=jax.ShapeDtypeStruct((NC, npad, 128), _F32),
        mesh=_sc_mesh(),
        scratch_types=[
            pltpu.VMEM((nch2, K), jnp.int32),
            pltpu.VMEM((nch2, K), jnp.int32),
            pltpu.VMEM((K, 128), _F32),
            pltpu.VMEM((K, 128), _F32),
            pltpu.VMEM_SHARED((npad, 128), _F32),
            pltpu.SemaphoreType.DMA,
            pltpu.SemaphoreType.DMA,
        ],
    )
    def gs_kernel(y, srcw, dstw, zeros_hbm, out,
                  sidx, didx, b0, b1, acc, sem0, sem1):
        c = lax.axis_index("c")
        s = lax.axis_index("s")
        wid = c * NS + s
        pltpu.sync_copy(zeros_hbm, b0)
        _zero_acc(b0, acc, s, rows, 128)
        plsc.subcore_barrier()

        def body(i, carry):
            t0 = 2 * i
            cp0 = pltpu.async_copy(y.at[sidx.at[t0]], b0, sem0)
            cp1 = pltpu.async_copy(y.at[sidx.at[t0 + 1]], b1, sem1)
            cp0.wait()
            pltpu.sync_copy(b0, acc.at[didx.at[t0]], add=True)
            cp1.wait()
            pltpu.sync_copy(b1, acc.at[didx.at[t0 + 1]], add=True)
            return carry

        for p in range(nph):
            pltpu.sync_copy(srcw.at[wid, pl.ds(p * nch2, nch2)], sidx)
            pltpu.sync_copy(dstw.at[wid, pl.ds(p * nch2, nch2)], didx)
            lax.fori_loop(0, nch2 // 2, body, 0)
        plsc.subcore_barrier()
        pltpu.sync_copy(acc.at[pl.ds(s * rows, rows)],
                        out.at[c, pl.ds(s * rows, rows)])

    return gs_kernel


def _pre_body(emb, pos, wpre, bpre, wg1, deg, y1):
    n = y1.shape[0]
    degsum = jnp.sum(deg[0, :n, :] + deg[1, :n, :], axis=1,
                     keepdims=True) + 1.0
    dis = lax.rsqrt(degsum)
    x = jnp.maximum(_dot(emb[...] + pos[...], wpre[...]) + bpre[...], 0.0)
    y1[...] = dis * _dot(x, wg1[...])


def _mid_body(acc, y, deg, bg, wg2, y2):
    n = y.shape[0]
    degsum = jnp.sum(deg[0, :n, :] + deg[1, :n, :], axis=1,
                     keepdims=True) + 1.0
    dis = lax.rsqrt(degsum)
    h = jnp.maximum(dis * (acc[0, :n, :] + acc[1, :n, :] + y[...])
                    + bg[...], 0.0)
    y2[...] = dis * _dot(h, wg2[...])


def _post_body(acc, y, deg, bg, batch2, wh, bh, out):
    n = y.shape[0]
    nseg = out.shape[0]
    degsum = jnp.sum(deg[0, :n, :] + deg[1, :n, :], axis=1,
                     keepdims=True) + 1.0
    dis = lax.rsqrt(degsum)
    h = jnp.maximum(dis * (acc[0, :n, :] + acc[1, :n, :] + y[...])
                    + bg[...], 0.0)
    seg_ids = lax.broadcasted_iota(jnp.int32, (1, nseg), 1)
    onehot = (batch2[...] == seg_ids).astype(_F32)
    seg = _dott(onehot, h)
    cnt = _dott(onehot, jnp.ones((n, 1), _F32))
    pooled = seg / jnp.maximum(cnt, 1.0)
    out[...] = jnp.maximum(_dot(pooled, wh[...]) + bh[...], 0.0)


def kernel(embeddings, positional_embeddings, edge_index, batch,
           W_pre, b_pre, W_g1, b_g1, W_g2, b_g2, W_head, b_head):
    n, d = embeddings.shape
    e = edge_index.shape[1]

    epw = -(-e // NW)
    nch = -(-(-(-epw // K)) // 4) * 4
    ep = nch * K
    npad = -(-(n + 1) // (NS * 8)) * (NS * 8)

    src = edge_index[0]
    dst = edge_index[1]
    pad_total = NW * ep - e
    pad_dst = n + jnp.arange(pad_total, dtype=dst.dtype) % (npad - n)
    pad_src = jnp.arange(pad_total, dtype=src.dtype) % (npad - n)
    src_p = jnp.concatenate([src, pad_src]).reshape(NW, nch, K)
    dst_p = jnp.concatenate([dst, pad_dst]).reshape(NW, nch, K)

    ones128 = jnp.zeros((K, d), _F32).at[:, 0].set(1.0)
    zeros128 = jnp.zeros((K, d), _F32)

    deg = _make_deg_kernel(npad, nch)(dst_p, ones128, zeros128)

    gs = _make_gs_kernel(n, npad, nch)

    y1 = pl.pallas_call(
        _pre_body,
        out_shape=jax.ShapeDtypeStruct((n, d), _F32),
    )(embeddings, positional_embeddings, W_pre, b_pre.reshape(1, d),
      W_g1, deg)

    acc1 = gs(y1, src_p, dst_p, zeros128)

    y2 = pl.pallas_call(
        _mid_body,
        out_shape=jax.ShapeDtypeStruct((n, d), _F32),
    )(acc1, y1, deg, b_g1.reshape(1, d), W_g2)

    acc2 = gs(y2, src_p, dst_p, zeros128)

    d_out = W_head.shape[1]
    out = pl.pallas_call(
        _post_body,
        out_shape=jax.ShapeDtypeStruct((16, d_out), _F32),
    )(acc2, y2, deg, b_g2.reshape(1, d), batch.reshape(n, 1),
      W_head, b_head.reshape(1, d_out))
    return out


# speedup vs baseline: 23.7169x; 1.2122x over previous
import functools
import jax
import jax.numpy as jnp
from jax import lax
from jax.experimental import pallas as pl
from jax.experimental.pallas import tpu as pltpu
from jax.experimental.pallas import tpu_sc as plsc

NC = 2
NS = 16
NW = NC * NS
K = 128

_F32 = jnp.float32
_HI = lax.Precision.HIGHEST


def _dot(a, b):
    return lax.dot_general(a, b, (((a.ndim - 1,), (0,)), ((), ())),
                           precision=_HI, preferred_element_type=_F32)


def _dott(a, b):
    return lax.dot_general(a, b, (((0,), (0,)), ((), ())),
                           precision=_HI, preferred_element_type=_F32)


def _sc_mesh():
    return plsc.VectorSubcoreMesh(core_axis_name="c", subcore_axis_name="s",
                                  num_cores=NC, num_subcores=NS)


def _zero_acc(zrows, acc, s, rows, width):
    base = s * rows
    nfull = rows // K
    for j in range(nfull):
        pltpu.sync_copy(zrows, acc.at[pl.ds(base + j * K, K)])
    rem = rows - nfull * K
    if rem:
        pltpu.sync_copy(zrows.at[pl.ds(0, rem)],
                        acc.at[pl.ds(base + nfull * K, rem)])


def _make_deg_kernel(npad, nch):
    rows = npad // NS

    @functools.partial(
        pl.kernel,
        out_type=jax.ShapeDtypeStruct((NC, npad, 128), _F32),
        mesh=_sc_mesh(),
        scratch_types=[
            pltpu.VMEM((nch, K), jnp.int32),
            pltpu.VMEM((K, 128), _F32),
            pltpu.VMEM_SHARED((npad, 128), _F32),
            pltpu.SemaphoreType.DMA,
        ],
    )
    def deg_kernel(dstw, ones_hbm, zeros_hbm, out, idx_v, buf, acc, sem):
        c = lax.axis_index("c")
        s = lax.axis_index("s")
        wid = c * NS + s
        pltpu.sync_copy(dstw.at[wid], idx_v)
        pltpu.sync_copy(zeros_hbm, buf)
        _zero_acc(buf, acc, s, rows, 128)
        plsc.subcore_barrier()
        pltpu.sync_copy(ones_hbm, buf)

        def fire(t, carry):
            pltpu.async_copy(buf, acc.at[idx_v.at[t]], sem, add=True)
            return carry

        lax.fori_loop(0, nch, fire, 0)

        def drain(t, carry):
            pltpu.make_async_copy(buf, acc.at[idx_v.at[t]], sem).wait()
            return carry

        lax.fori_loop(0, nch, drain, 0)
        plsc.subcore_barrier()
        pltpu.sync_copy(acc.at[pl.ds(s * rows, rows)],
                        out.at[c, pl.ds(s * rows, rows)])

    return deg_kernel


def _make_gs_kernel(n, npad, nch):
    rows = npad // NS
    nph = 2
    nch2 = nch // nph

    @functools.partial(
        pl.kernel,
        out_type=jax.ShapeDtypeStruct((NC, npad, 128), _F32),
        mesh=_sc_mesh(),
        scratch_types=[
            pltpu.VMEM((nch2, K), jnp.int32),
            pltpu.VMEM((nch2, K), jnp.int32),
            pltpu.VMEM((K, 128), _F32),
            pltpu.VMEM((K, 128), _F32),
            pltpu.VMEM_SHARED((npad, 128), _F32),
            pltpu.SemaphoreType.DMA,
            pltpu.SemaphoreType.DMA,
            pltpu.SemaphoreType.DMA,
            pltpu.SemaphoreType.DMA,
        ],
    )
    def gs_kernel(y, srcw, dstw, zeros_hbm, out,
                  sidx, didx, b0, b1, acc, g0, g1, s0, s1):
        c = lax.axis_index("c")
        s = lax.axis_index("s")
        wid = c * NS + s
        pltpu.sync_copy(zeros_hbm, b0)
        _zero_acc(b0, acc, s, rows, 128)
        plsc.subcore_barrier()

        def sg(t, buf, sem):
            pltpu.async_copy(y.at[sidx.at[t]], buf, sem)

        def wg(t, buf, sem):
            pltpu.make_async_copy(y.at[sidx.at[t]], buf, sem).wait()

        def ss(t, buf, sem):
            pltpu.async_copy(buf, acc.at[didx.at[t]], sem, add=True)

        def ws(t, buf, sem):
            pltpu.make_async_copy(buf, acc.at[didx.at[t]], sem).wait()

        def body(i, carry):
            t = 2 * i
            ws(t - 2, b0, s0)
            sg(t, b0, g0)
            wg(t - 1, b1, g1)
            ss(t - 1, b1, s1)
            ws(t - 1, b1, s1)
            sg(t + 1, b1, g1)
            wg(t, b0, g0)
            ss(t, b0, s0)
            return carry

        for p in range(nph):
            pltpu.sync_copy(srcw.at[wid, pl.ds(p * nch2, nch2)], sidx)
            pltpu.sync_copy(dstw.at[wid, pl.ds(p * nch2, nch2)], didx)
            sg(0, b0, g0)
            sg(1, b1, g1)
            wg(0, b0, g0)
            ss(0, b0, s0)
            lax.fori_loop(1, nch2 // 2, body, 0)
            wg(nch2 - 1, b1, g1)
            ss(nch2 - 1, b1, s1)
            ws(nch2 - 2, b0, s0)
            ws(nch2 - 1, b1, s1)
        plsc.subcore_barrier()
        pltpu.sync_copy(acc.at[pl.ds(s * rows, rows)],
                        out.at[c, pl.ds(s * rows, rows)])

    return gs_kernel


def _pre_body(emb, pos, wpre, bpre, wg1, deg, y1, dis2):
    n = y1.shape[0]
    degsum = jnp.sum(deg[0, :n, :] + deg[1, :n, :], axis=1,
                     keepdims=True) + 1.0
    dis = lax.rsqrt(degsum)
    dis2[...] = dis
    x = jnp.maximum(_dot(emb[...] + pos[...], wpre[...]) + bpre[...], 0.0)
    y1[...] = dis * _dot(x, wg1[...])


def _mid_body(acc, y, dis2, bg, wg2, y2):
    n = y.shape[0]
    dis = dis2[...]
    h = jnp.maximum(dis * (acc[0, :n, :] + acc[1, :n, :] + y[...])
                    + bg[...], 0.0)
    y2[...] = dis * _dot(h, wg2[...])


def _post_body(acc, y, dis2, bg, batch2, wh, bh, out):
    n = y.shape[0]
    nseg = out.shape[0]
    dis = dis2[...]
    h = jnp.maximum(dis * (acc[0, :n, :] + acc[1, :n, :] + y[...])
                    + bg[...], 0.0)
    seg_ids = lax.broadcasted_iota(jnp.int32, (1, nseg), 1)
    onehot = (batch2[...] == seg_ids).astype(_F32)
    seg = _dott(onehot, h)
    cnt = _dott(onehot, jnp.ones((n, 1), _F32))
    pooled = seg / jnp.maximum(cnt, 1.0)
    out[...] = jnp.maximum(_dot(pooled, wh[...]) + bh[...], 0.0)


def kernel(embeddings, positional_embeddings, edge_index, batch,
           W_pre, b_pre, W_g1, b_g1, W_g2, b_g2, W_head, b_head):
    n, d = embeddings.shape
    e = edge_index.shape[1]

    epw = -(-e // NW)
    nch = -(-(-(-epw // K)) // 4) * 4
    ep = nch * K
    npad = -(-(n + 1) // (NS * 8)) * (NS * 8)

    src = edge_index[0]
    dst = edge_index[1]
    pad_total = NW * ep - e
    pad_dst = n + jnp.arange(pad_total, dtype=dst.dtype) % (npad - n)
    pad_src = jnp.arange(pad_total, dtype=src.dtype) % (npad - n)
    src_p = jnp.concatenate([src, pad_src]).reshape(NW, nch, K)
    dst_p = jnp.concatenate([dst, pad_dst]).reshape(NW, nch, K)

    ones128 = jnp.zeros((K, d), _F32).at[:, 0].set(1.0)
    zeros128 = jnp.zeros((K, d), _F32)

    deg = _make_deg_kernel(npad, nch)(dst_p, ones128, zeros128)

    gs = _make_gs_kernel(n, npad, nch)

    y1, dis2 = pl.pallas_call(
        _pre_body,
        out_shape=[jax.ShapeDtypeStruct((n, d), _F32),
                   jax.ShapeDtypeStruct((n, 1), _F32)],
    )(embeddings, positional_embeddings, W_pre, b_pre.reshape(1, d),
      W_g1, deg)

    acc1 = gs(y1, src_p, dst_p, zeros128)

    y2 = pl.pallas_call(
        _mid_body,
        out_shape=jax.ShapeDtypeStruct((n, d), _F32),
    )(acc1, y1, dis2, b_g1.reshape(1, d), W_g2)

    acc2 = gs(y2, src_p, dst_p, zeros128)

    d_out = W_head.shape[1]
    out = pl.pallas_call(
        _post_body,
        out_shape=jax.ShapeDtypeStruct((16, d_out), _F32),
    )(acc2, y2, dis2, b_g2.reshape(1, d), batch.reshape(n, 1),
      W_head, b_head.reshape(1, d_out))
    return out


# speedup vs baseline: 24.6618x; 1.0398x over previous
import functools
import jax
import jax.numpy as jnp
from jax import lax
from jax.experimental import pallas as pl
from jax.experimental.pallas import tpu as pltpu
from jax.experimental.pallas import tpu_sc as plsc

NC = 2
NS = 16
NW = NC * NS
K = 128

_F32 = jnp.float32
_HI = lax.Precision.HIGHEST


def _dot(a, b):
    return lax.dot_general(a, b, (((a.ndim - 1,), (0,)), ((), ())),
                           precision=_HI, preferred_element_type=_F32)


def _dott(a, b):
    return lax.dot_general(a, b, (((0,), (0,)), ((), ())),
                           precision=_HI, preferred_element_type=_F32)


def _sc_mesh():
    return plsc.VectorSubcoreMesh(core_axis_name="c", subcore_axis_name="s",
                                  num_cores=NC, num_subcores=NS)


def _zero_acc(zrows, acc, s, rows, width):
    base = s * rows
    nfull = rows // K
    for j in range(nfull):
        pltpu.sync_copy(zrows, acc.at[pl.ds(base + j * K, K)])
    rem = rows - nfull * K
    if rem:
        pltpu.sync_copy(zrows.at[pl.ds(0, rem)],
                        acc.at[pl.ds(base + nfull * K, rem)])


def _make_deg_kernel(npad, nch):
    rows = npad // NS

    @functools.partial(
        pl.kernel,
        out_type=jax.ShapeDtypeStruct((NC, npad, 128), _F32),
        mesh=_sc_mesh(),
        scratch_types=[
            pltpu.VMEM((nch, K), jnp.int32),
            pltpu.VMEM((K, 128), _F32),
            pltpu.VMEM_SHARED((npad, 128), _F32),
            pltpu.SemaphoreType.DMA,
        ],
    )
    def deg_kernel(dstw, ones_hbm, zeros_hbm, out, idx_v, buf, acc, sem):
        c = lax.axis_index("c")
        s = lax.axis_index("s")
        wid = c * NS + s
        pltpu.sync_copy(dstw.at[wid], idx_v)
        pltpu.sync_copy(zeros_hbm, buf)
        _zero_acc(buf, acc, s, rows, 128)
        plsc.subcore_barrier()
        pltpu.sync_copy(ones_hbm, buf)

        def fire(t, carry):
            pltpu.async_copy(buf, acc.at[idx_v.at[t]], sem, add=True)
            return carry

        lax.fori_loop(0, nch, fire, 0)

        def drain(t, carry):
            pltpu.make_async_copy(buf, acc.at[idx_v.at[t]], sem).wait()
            return carry

        lax.fori_loop(0, nch, drain, 0)
        plsc.subcore_barrier()
        pltpu.sync_copy(acc.at[pl.ds(s * rows, rows)],
                        out.at[c, pl.ds(s * rows, rows)])

    return deg_kernel


def _make_gs_kernel(n, npad, nch):
    rows = npad // NS
    nph = 2
    nch2 = nch // nph

    @functools.partial(
        pl.kernel,
        out_type=jax.ShapeDtypeStruct((NC, npad, 128), _F32),
        mesh=_sc_mesh(),
        scratch_types=[
            pltpu.VMEM((nch2, K), jnp.int32),
            pltpu.VMEM((nch2, K), jnp.int32),
            pltpu.VMEM((K, 128), _F32),
            pltpu.VMEM((K, 128), _F32),
            pltpu.VMEM_SHARED((npad, 128), _F32),
            pltpu.SemaphoreType.DMA,
            pltpu.SemaphoreType.DMA,
            pltpu.SemaphoreType.DMA,
            pltpu.SemaphoreType.DMA,
        ],
    )
    def gs_kernel(y, srcw, dstw, zeros_hbm, out,
                  sidx, didx, b0, b1, acc, g0, g1, s0, s1):
        c = lax.axis_index("c")
        s = lax.axis_index("s")
        wid = c * NS + s
        pltpu.sync_copy(zeros_hbm, b0)
        _zero_acc(b0, acc, s, rows, 128)
        plsc.subcore_barrier()

        def sg(t, buf, sem):
            pltpu.async_copy(y.at[sidx.at[t]], buf, sem)

        def wg(t, buf, sem):
            pltpu.make_async_copy(y.at[sidx.at[t]], buf, sem).wait()

        def ss(t, buf, sem):
            pltpu.async_copy(buf, acc.at[didx.at[t]], sem, add=True)

        def ws(t, buf, sem):
            pltpu.make_async_copy(buf, acc.at[didx.at[t]], sem).wait()

        def body(i, carry):
            t = 2 * i
            ws(t - 2, b0, s0)
            sg(t, b0, g0)
            wg(t - 1, b1, g1)
            ss(t - 1, b1, s1)
            ws(t - 1, b1, s1)
            sg(t + 1, b1, g1)
            wg(t, b0, g0)
            ss(t, b0, s0)
            return carry

        for p in range(nph):
            pltpu.sync_copy(srcw.at[wid, pl.ds(p * nch2, nch2)], sidx)
            pltpu.sync_copy(dstw.at[wid, pl.ds(p * nch2, nch2)], didx)
            sg(0, b0, g0)
            sg(1, b1, g1)
            wg(0, b0, g0)
            ss(0, b0, s0)
            lax.fori_loop(1, nch2 // 2, body, 0)
            wg(nch2 - 1, b1, g1)
            ss(nch2 - 1, b1, s1)
            ws(nch2 - 2, b0, s0)
            ws(nch2 - 1, b1, s1)
        plsc.subcore_barrier()
        pltpu.sync_copy(acc.at[pl.ds(s * rows, rows)],
                        out.at[c, pl.ds(s * rows, rows)])

    return gs_kernel


def _prea_body(emb, pos, wpre, bpre, wg1, xw1):
    x = jnp.maximum(_dot(emb[...] + pos[...], wpre[...]) + bpre[...], 0.0)
    xw1[...] = _dot(x, wg1[...])


def _preb_body(xw1, deg, y1, dis2):
    n = y1.shape[0]
    degsum = jnp.sum(deg[0, :n, :] + deg[1, :n, :], axis=1,
                     keepdims=True) + 1.0
    dis = lax.rsqrt(degsum)
    dis2[...] = dis
    y1[...] = dis * xw1[...]


def _mid_body(acc, y, dis2, bg, wg2, y2):
    n = y.shape[0]
    dis = dis2[...]
    h = jnp.maximum(dis * (acc[0, :n, :] + acc[1, :n, :] + y[...])
                    + bg[...], 0.0)
    y2[...] = dis * _dot(h, wg2[...])


def _post_body(acc, y, dis2, bg, batch2, wh, bh, out):
    n = y.shape[0]
    nseg = out.shape[0]
    dis = dis2[...]
    h = jnp.maximum(dis * (acc[0, :n, :] + acc[1, :n, :] + y[...])
                    + bg[...], 0.0)
    seg_ids = lax.broadcasted_iota(jnp.int32, (1, nseg), 1)
    onehot = (batch2[...] == seg_ids).astype(_F32)
    seg = _dott(onehot, h)
    cnt = _dott(onehot, jnp.ones((n, 1), _F32))
    pooled = seg / jnp.maximum(cnt, 1.0)
    out[...] = jnp.maximum(_dot(pooled, wh[...]) + bh[...], 0.0)


def kernel(embeddings, positional_embeddings, edge_index, batch,
           W_pre, b_pre, W_g1, b_g1, W_g2, b_g2, W_head, b_head):
    n, d = embeddings.shape
    e = edge_index.shape[1]

    epw = -(-e // NW)
    nch = -(-(-(-epw // K)) // 4) * 4
    ep = nch * K
    npad = -(-(n + 1) // (NS * 8)) * (NS * 8)

    src = edge_index[0]
    dst = edge_index[1]
    pad_total = NW * ep - e
    pad_dst = n + jnp.arange(pad_total, dtype=dst.dtype) % (npad - n)
    pad_src = jnp.arange(pad_total, dtype=src.dtype) % (npad - n)
    src_p = jnp.concatenate([src, pad_src]).reshape(NW, nch, K)
    dst_p = jnp.concatenate([dst, pad_dst]).reshape(NW, nch, K)

    ones128 = jnp.zeros((K, d), _F32).at[:, 0].set(1.0)
    zeros128 = jnp.zeros((K, d), _F32)

    deg = _make_deg_kernel(npad, nch)(dst_p, ones128, zeros128)

    gs = _make_gs_kernel(n, npad, nch)

    xw1 = pl.pallas_call(
        _prea_body,
        out_shape=jax.ShapeDtypeStruct((n, d), _F32),
    )(embeddings, positional_embeddings, W_pre, b_pre.reshape(1, d), W_g1)

    y1, dis2 = pl.pallas_call(
        _preb_body,
        out_shape=[jax.ShapeDtypeStruct((n, d), _F32),
                   jax.ShapeDtypeStruct((n, 1), _F32)],
    )(xw1, deg)

    acc1 = gs(y1, src_p, dst_p, zeros128)

    y2 = pl.pallas_call(
        _mid_body,
        out_shape=jax.ShapeDtypeStruct((n, d), _F32),
    )(acc1, y1, dis2, b_g1.reshape(1, d), W_g2)

    acc2 = gs(y2, src_p, dst_p, zeros128)

    d_out = W_head.shape[1]
    out = pl.pallas_call(
        _post_body,
        out_shape=jax.ShapeDtypeStruct((16, d_out), _F32),
    )(acc2, y2, dis2, b_g2.reshape(1, d), batch.reshape(n, 1),
      W_head, b_head.reshape(1, d_out))
    return out
